# Initial kernel scaffold; baseline (speedup 1.0000x reference)
import functools

import jax
import jax.numpy as jnp
from jax import lax
from jax.experimental import pallas as pl
from jax.experimental.pallas import tpu as pltpu
from jax.experimental.pallas import tpu_sc as plsc

N = 10000
T = 8
D = 128
E = 320000

NC = 2
NS = 16
CH = 128
NCH = 157
EP = NS * NCH * CH
NP = 10240
RPT = N // NS
DPT = NP // NS

_SC_MESH = dict(core_axis_name="c", subcore_axis_name="s")


def _deg_body(dst_hbm, deg_hbm, dst_v, ones_v, buf_v, acc_sh):
    c = lax.axis_index("c")
    s = lax.axis_index("s")
    pltpu.sync_copy(dst_hbm.at[s], dst_v)
    for i in range(CH // 16):
        ones_v[pl.ds(i * 16, 16)] = jnp.ones((16,), jnp.float32)
    for i in range(DPT // 16):
        buf_v[pl.ds(i * 16, 16)] = jnp.zeros((16,), jnp.float32)
    pltpu.sync_copy(buf_v, acc_sh.at[pl.ds(s * DPT, DPT)])
    plsc.subcore_barrier()

    @pl.loop(c, NCH, step=2)
    def _edges(j):
        pltpu.sync_copy(ones_v, acc_sh.at[dst_v.at[j]], add=True)

    plsc.subcore_barrier()
    pltpu.sync_copy(acc_sh.at[pl.ds(s * DPT, DPT)],
                    deg_hbm.at[c, pl.ds(s * DPT, DPT)])


def _deg_call(dst_p):
    return pl.kernel(
        _deg_body,
        out_type=jax.ShapeDtypeStruct((NC, NP), jnp.float32),
        mesh=plsc.VectorSubcoreMesh(**_SC_MESH),
        scratch_types=[
            pltpu.VMEM((NCH, CH), jnp.int32),
            pltpu.VMEM((CH,), jnp.float32),
            pltpu.VMEM((DPT,), jnp.float32),
            pltpu.VMEM_SHARED((NP,), jnp.float32),
        ],
    )(dst_p)


_BN_F = 2000


def _feat_body(x_ref, w_ref, deg_ref, u_ref, dis_ref):
    xb = x_ref[:, 0, :]
    deg = 1.0 + deg_ref[0, :] + deg_ref[1, :]
    dis = lax.rsqrt(deg)
    xw = jnp.dot(xb, w_ref[...], preferred_element_type=jnp.float32)
    u_ref[0] = xw * dis[:, None]
    dis_ref[...] = dis[:, None]


def _feat_call(x, W_conv, deg2):
    return pl.pallas_call(
        _feat_body,
        grid=(T, N // _BN_F),
        in_specs=[
            pl.BlockSpec((_BN_F, 1, D), lambda t, j: (j, t, 0)),
            pl.BlockSpec((D, D), lambda t, j: (0, 0)),
            pl.BlockSpec((NC, _BN_F), lambda t, j: (0, j)),
        ],
        out_specs=[
            pl.BlockSpec((1, _BN_F, D), lambda t, j: (t, j, 0)),
            pl.BlockSpec((_BN_F, 1), lambda t, j: (j, 0)),
        ],
        out_shape=[
            jax.ShapeDtypeStruct((T, N, D), jnp.float32),
            jax.ShapeDtypeStruct((N, 1), jnp.float32),
        ],
    )(x, W_conv, deg2)


def _conv_body(u_hbm, src_hbm, dst_hbm, out_hbm, src_v, dst_v, rows_v, sem,
               acc_sh):
    c = lax.axis_index("c")
    s = lax.axis_index("s")
    pltpu.sync_copy(dst_hbm.at[s], dst_v)
    for tl in range(T // NC):
        t = c * (T // NC) + tl
        pltpu.sync_copy(src_hbm.at[t, s], src_v)
        pltpu.sync_copy(u_hbm.at[pl.ds(t * N + s * RPT, RPT)],
                        acc_sh.at[pl.ds(s * RPT, RPT)])
        plsc.subcore_barrier()

        @pl.loop(0, NCH)
        def _edges(j):
            pltpu.async_copy(u_hbm.at[src_v.at[j]], rows_v, sem).wait()
            pltpu.sync_copy(rows_v, acc_sh.at[dst_v.at[j]], add=True)

        plsc.subcore_barrier()
        pltpu.sync_copy(acc_sh.at[pl.ds(s * RPT, RPT)],
                        out_hbm.at[pl.ds(t * N + s * RPT, RPT)])
        plsc.subcore_barrier()


def _conv_call(u_flat, src_abs, dst_p):
    return pl.kernel(
        _conv_body,
        out_type=jax.ShapeDtypeStruct((T * N, D), jnp.float32),
        mesh=plsc.VectorSubcoreMesh(**_SC_MESH),
        scratch_types=[
            pltpu.VMEM((NCH, CH), jnp.int32),
            pltpu.VMEM((NCH, CH), jnp.int32),
            pltpu.VMEM((CH, D), jnp.float32),
            pltpu.SemaphoreType.DMA,
            pltpu.VMEM_SHARED((NP, D), jnp.float32),
        ],
    )(u_flat, src_abs, dst_p)


_BN_R = 2000


def _rnn_body(i2h_ref, dis_ref, bc_ref, wl_ref, bl_ref, out_ref):
    dis = dis_ref[...]
    wl = wl_ref[...]
    h = jnp.zeros((_BN_R, D), jnp.float32)
    for t in range(T):
        a = (i2h_ref[t] * dis + bc_ref[...] + bl_ref[...]
             + lax.dot_general(h, wl, (((1,), (1,)), ((), ())),
                               preferred_element_type=jnp.float32))
        h = jnp.tanh(a)
        out_ref[:, t, :] = h


def _rnn_call(i2h, dis, b_conv, W_lin, b_lin):
    return pl.pallas_call(
        _rnn_body,
        grid=(N // _BN_R,),
        in_specs=[
            pl.BlockSpec((T, _BN_R, D), lambda j: (0, j, 0)),
            pl.BlockSpec((_BN_R, 1), lambda j: (j, 0)),
            pl.BlockSpec((1, D), lambda j: (0, 0)),
            pl.BlockSpec((D, D), lambda j: (0, 0)),
            pl.BlockSpec((1, D), lambda j: (0, 0)),
        ],
        out_specs=pl.BlockSpec((_BN_R, T, D), lambda j: (j, 0, 0)),
        out_shape=jax.ShapeDtypeStruct((N, T, D), jnp.float32),
    )(i2h, dis, b_conv, W_lin, b_lin)


def kernel(x, edge_index, W_conv, b_conv, W_lin, b_lin):
    src = edge_index[0].astype(jnp.int32)
    dst = edge_index[1].astype(jnp.int32)
    pad = EP - E
    pad_idx = jnp.arange(pad, dtype=jnp.int32)
    src_p = jnp.concatenate([src, pad_idx % 128]).reshape(NS, NCH, CH)
    dst_p = jnp.concatenate([dst, N + pad_idx % (NP - N)]).reshape(NS, NCH, CH)
    offs = (jnp.arange(T, dtype=jnp.int32) * N)[:, None, None, None]
    src_abs = src_p[None] + offs

    deg2 = _deg_call(dst_p)
    u, dis = _feat_call(x, W_conv, deg2)
    i2h = _conv_call(u.reshape(T * N, D), src_abs, dst_p)
    return _rnn_call(i2h.reshape(T, N, D), dis,
                     b_conv.reshape(1, D), W_lin, b_lin)


# speedup vs baseline: 20.3986x; 20.3986x over previous
import functools

import jax
import jax.numpy as jnp
from jax import lax
from jax.experimental import pallas as pl
from jax.experimental.pallas import tpu as pltpu
from jax.experimental.pallas import tpu_sc as plsc

N = 10000
T = 8
D = 128
E = 320000

NC = 2
NS = 16
CH = 128
NCH = 160
SB = 40
NSB = NCH // SB
EP = NS * NCH * CH
NP = 10240
RPT = 632
RPT_LAST = N - RPT * (NS - 1)
DPT = NP // NS

_SC_MESH = dict(core_axis_name="c", subcore_axis_name="s")


def _deg_body(dst_hbm, deg_hbm, dst_v, ones_v, buf_v, acc_sh):
    c = lax.axis_index("c")
    s = lax.axis_index("s")
    pltpu.sync_copy(dst_hbm.at[s], dst_v)
    for i in range(CH // 16):
        ones_v[pl.ds(i * 16, 16)] = jnp.ones((16,), jnp.float32)
    for i in range(DPT // 16):
        buf_v[pl.ds(i * 16, 16)] = jnp.zeros((16,), jnp.float32)
    pltpu.sync_copy(buf_v, acc_sh.at[pl.ds(s * DPT, DPT)])
    plsc.subcore_barrier()

    for sb in range(NSB):
        @pl.loop(c, SB, step=2)
        def _edges(j):
            pltpu.sync_copy(ones_v, acc_sh.at[dst_v.at[sb, j]], add=True)

    plsc.subcore_barrier()
    pltpu.sync_copy(acc_sh.at[pl.ds(s * DPT, DPT)],
                    deg_hbm.at[c, pl.ds(s * DPT, DPT)])


def _deg_call(dst_p):
    return pl.kernel(
        _deg_body,
        out_type=jax.ShapeDtypeStruct((NC, NP), jnp.float32),
        mesh=plsc.VectorSubcoreMesh(**_SC_MESH),
        scratch_types=[
            pltpu.VMEM((NSB, SB, CH), jnp.int32),
            pltpu.VMEM((CH,), jnp.float32),
            pltpu.VMEM((DPT,), jnp.float32),
            pltpu.VMEM_SHARED((NP,), jnp.float32),
        ],
    )(dst_p)


_BN_F = 2000


def _feat_body(x_ref, w_ref, deg_ref, u_ref, dis_ref):
    xb = x_ref[...]
    deg = 1.0 + deg_ref[:, 0] + deg_ref[:, 1]
    dis = lax.rsqrt(deg)
    w = w_ref[...]
    for t in range(T):
        u_ref[t] = jnp.dot(xb[:, t, :], w,
                           preferred_element_type=jnp.float32) * dis[:, None]
    dis_ref[...] = dis[:, None]


def _feat_call(x, W_conv, deg2t):
    return pl.pallas_call(
        _feat_body,
        grid=(N // _BN_F,),
        in_specs=[
            pl.BlockSpec((_BN_F, T, D), lambda j: (j, 0, 0)),
            pl.BlockSpec((D, D), lambda j: (0, 0)),
            pl.BlockSpec((_BN_F, NC), lambda j: (j, 0)),
        ],
        out_specs=[
            pl.BlockSpec((T, _BN_F, D), lambda j: (0, j, 0)),
            pl.BlockSpec((_BN_F, 1), lambda j: (j, 0)),
        ],
        out_shape=[
            jax.ShapeDtypeStruct((T, N, D), jnp.float32),
            jax.ShapeDtypeStruct((N, 1), jnp.float32),
        ],
    )(x, W_conv, deg2t)


def _conv_body(u_hbm, src_hbm, dst_hbm, out_hbm, src_v, dst_v, rows_v, sem,
               acc_sh):
    c = lax.axis_index("c")
    s = lax.axis_index("s")
    for tl in range(T // NC):
        t = c * (T // NC) + tl
        @pl.when(s < NS - 1)
        def _():
            pltpu.sync_copy(u_hbm.at[pl.ds(t * N + s * RPT, RPT)],
                            acc_sh.at[pl.ds(s * RPT, RPT)])

        @pl.when(s == NS - 1)
        def _():
            pltpu.sync_copy(u_hbm.at[pl.ds(t * N + (NS - 1) * RPT, RPT_LAST)],
                            acc_sh.at[pl.ds((NS - 1) * RPT, RPT_LAST)])

        plsc.subcore_barrier()

        for sb in range(NSB):
            pltpu.sync_copy(src_hbm.at[t].at[s].at[sb], src_v)
            pltpu.sync_copy(dst_hbm.at[s].at[sb], dst_v)
            pltpu.async_copy(u_hbm.at[src_v.at[0]], rows_v.at[0], sem.at[0])

            @pl.loop(0, SB)
            def _edges(j):
                p = lax.rem(j, 2)
                q = lax.rem(j + 1, 2)
                pltpu.make_async_copy(u_hbm.at[src_v.at[j]], rows_v.at[p],
                                      sem.at[p]).wait()

                @pl.when(j + 1 < SB)
                def _():
                    pltpu.async_copy(u_hbm.at[src_v.at[j + 1]], rows_v.at[q],
                                     sem.at[q])

                pltpu.sync_copy(rows_v.at[p], acc_sh.at[dst_v.at[j]],
                                add=True)

        plsc.subcore_barrier()

        @pl.when(s < NS - 1)
        def _():
            pltpu.sync_copy(acc_sh.at[pl.ds(s * RPT, RPT)],
                            out_hbm.at[pl.ds(t * N + s * RPT, RPT)])

        @pl.when(s == NS - 1)
        def _():
            pltpu.sync_copy(acc_sh.at[pl.ds((NS - 1) * RPT, RPT_LAST)],
                            out_hbm.at[pl.ds(t * N + (NS - 1) * RPT, RPT_LAST)])

        plsc.subcore_barrier()


def _conv_call(u_flat, src_abs, dst_p):
    return pl.kernel(
        _conv_body,
        out_type=jax.ShapeDtypeStruct((T * N, D), jnp.float32),
        mesh=plsc.VectorSubcoreMesh(**_SC_MESH),
        scratch_types=[
            pltpu.VMEM((SB, CH), jnp.int32),
            pltpu.VMEM((SB, CH), jnp.int32),
            pltpu.VMEM((2, CH, D), jnp.float32),
            pltpu.SemaphoreType.DMA((2,)),
            pltpu.VMEM_SHARED((NP, D), jnp.float32),
        ],
    )(u_flat, src_abs, dst_p)


_BN_R = 2000


def _rnn_body(i2h_ref, dis_ref, bc_ref, wl_ref, bl_ref, out_ref):
    dis = dis_ref[...]
    wl = wl_ref[...]
    h = jnp.zeros((_BN_R, D), jnp.float32)
    for t in range(T):
        a = (i2h_ref[t] * dis + bc_ref[...] + bl_ref[...]
             + lax.dot_general(h, wl, (((1,), (1,)), ((), ())),
                               preferred_element_type=jnp.float32))
        h = jnp.tanh(a)
        out_ref[:, t, :] = h


def _rnn_call(i2h, dis, b_conv, W_lin, b_lin):
    return pl.pallas_call(
        _rnn_body,
        grid=(N // _BN_R,),
        in_specs=[
            pl.BlockSpec((T, _BN_R, D), lambda j: (0, j, 0)),
            pl.BlockSpec((_BN_R, 1), lambda j: (j, 0)),
            pl.BlockSpec((1, D), lambda j: (0, 0)),
            pl.BlockSpec((D, D), lambda j: (0, 0)),
            pl.BlockSpec((1, D), lambda j: (0, 0)),
        ],
        out_specs=pl.BlockSpec((_BN_R, T, D), lambda j: (j, 0, 0)),
        out_shape=jax.ShapeDtypeStruct((N, T, D), jnp.float32),
    )(i2h, dis, b_conv.reshape(1, D), W_lin, b_lin.reshape(1, D))


def kernel(x, edge_index, W_conv, b_conv, W_lin, b_lin):
    src = edge_index[0].astype(jnp.int32)
    dst = edge_index[1].astype(jnp.int32)
    pad = EP - E
    pad_idx = jnp.arange(pad, dtype=jnp.int32)
    src_p = jnp.concatenate([src, pad_idx % 128]).reshape(NS, NSB, SB, CH)
    dst_p = jnp.concatenate([dst, N + pad_idx % (NP - N)]
                            ).reshape(NS, NSB, SB, CH)
    offs = (jnp.arange(T, dtype=jnp.int32) * N)[:, None, None, None, None]
    src_abs = src_p[None] + offs

    deg2 = _deg_call(dst_p)
    u, dis = _feat_call(x, W_conv, deg2.T)
    i2h = _conv_call(u.reshape(T * N, D), src_abs, dst_p)
    return _rnn_call(i2h.reshape(T, N, D), dis, b_conv, W_lin, b_lin)
